# baseline (speedup 1.0000x reference)
import functools

import jax
import jax.numpy as jnp
from jax import lax
from jax.experimental import pallas as pl
from jax.experimental.pallas import tpu as pltpu
from jax.experimental.pallas import tpu_sc as plsc

B, S, C, P = 64, 32, 8192, 8
N = B * S
R = 256
GRID = N // R
NEG_INF = float("-inf")

L = 16
T_SC = 16
PPT = N * P // T_SC


def _fused_body(x_ref, t_ref, loss_ref, corr_ref, preds_ref):
    i = pl.program_id(0)
    x = x_ref[...]
    t = t_ref[...]
    iota = lax.broadcasted_iota(jnp.int32, (R, C), 1)

    xw = x
    idxs, vals = [], []
    for _ in range(P):
        m = jnp.max(xw, axis=1, keepdims=True)
        cand = jnp.where(xw == m, iota, C)
        idx = jnp.min(cand, axis=1, keepdims=True)
        idxs.append(idx)
        vals.append(m)
        xw = jnp.where(cand == idx, NEG_INF, xw)
    preds_ref[...] = jnp.concatenate(idxs, axis=1)

    rmax = vals[0]
    sumexp = jnp.sum(jnp.exp(xw - rmax), axis=1, keepdims=True)
    for k in range(P):
        sumexp = sumexp + jnp.exp(vals[k] - rmax)
    lse = jnp.log(sumexp) + rmax

    tacc = jnp.zeros((R, C), jnp.float32)
    for j in range(P):
        tj = t[:, j:j + 1]
        tacc = tacc + jnp.where(iota == tj, x, 0.0)
    tsum = jnp.sum(tacc, axis=1, keepdims=True)
    loss_blk = jnp.sum(lse - tsum / jnp.float32(P))

    corr_blk = jnp.float32(0.0)
    for k in range(P):
        mk = jnp.zeros((R, 1), jnp.bool_)
        for j in range(P):
            mk = mk | (idxs[k] == t[:, j:j + 1])
        corr_blk = corr_blk + jnp.sum(jnp.where(mk, 1.0, 0.0))

    @pl.when(i == 0)
    def _():
        loss_ref[...] = jnp.zeros_like(loss_ref)
        corr_ref[...] = jnp.zeros_like(corr_ref)

    loss_ref[...] += loss_blk
    corr_ref[...] += corr_blk


@jax.jit
def _run_tc(x2, t2):
    return pl.pallas_call(
        _fused_body,
        grid=(GRID,),
        in_specs=[
            pl.BlockSpec((R, C), lambda i: (i, 0)),
            pl.BlockSpec((R, P), lambda i: (i, 0)),
        ],
        out_specs=[
            pl.BlockSpec((1, 1), lambda i: (0, 0)),
            pl.BlockSpec((1, 1), lambda i: (0, 0)),
            pl.BlockSpec((R, P), lambda i: (i, 0)),
        ],
        out_shape=[
            jax.ShapeDtypeStruct((1, 1), jnp.float32),
            jax.ShapeDtypeStruct((1, 1), jnp.float32),
            jax.ShapeDtypeStruct((N, P), jnp.int32),
        ],
    )(x2, t2)


@functools.partial(
    pl.kernel,
    mesh=plsc.VectorSubcoreMesh(core_axis_name="c", subcore_axis_name="s",
                                num_cores=1),
    out_type=jax.ShapeDtypeStruct((C,), jnp.float32),
    scratch_types=[
        pltpu.VMEM((PPT,), jnp.int32),
        pltpu.VMEM((8, 128), jnp.int32),
        pltpu.VMEM((128,), jnp.float32),
        pltpu.VMEM((C,), jnp.float32),
        pltpu.VMEM_SHARED((C,), jnp.float32),
        pltpu.SemaphoreType.DMA,
    ],
)
def _sc_hist(preds_hbm, out_hbm, pidx_v, pidx2_v, ones_v, z_v, shared, sem):
    wid = lax.axis_index("s")
    base = wid * PPT
    pltpu.sync_copy(preds_hbm.at[pl.ds(base, PPT)], pidx_v)
    one16 = jnp.ones((L,), jnp.float32)
    zero16 = jnp.zeros((L,), jnp.float32)
    for k in range(8):
        ones_v[pl.ds(k * L, L)] = one16
    for k in range(PPT // L):
        pidx2_v[k // 8, pl.ds((k % 8) * L, L)] = pidx_v[pl.ds(k * L, L)]

    @pl.when(wid == 0)
    def _():
        for k in range(C // L):
            z_v[pl.ds(k * L, L)] = zero16
        pltpu.sync_copy(z_v, shared)

    plsc.subcore_barrier()
    for b in range(8):
        pltpu.sync_copy(ones_v, shared.at[pidx2_v.at[b]], add=True)
    plsc.subcore_barrier()

    @pl.when(wid == 0)
    def _():
        pltpu.sync_copy(shared, out_hbm)


def kernel(output, target):
    bb, ss, cc = output.shape
    x2 = output.reshape(N, C)
    t2 = target.reshape(N, P)
    loss_sum, correct, preds = _run_tc(x2, t2)
    counts = _sc_hist(preds.reshape(N * P))
    loss = loss_sum[0, 0] / jnp.float32(N)
    acc = correct[0, 0] / jnp.float32(N * P) * 100.0
    p_counts = (counts / counts.sum() * 100.0).astype(jnp.int32)
    prompt_id_preds = preds.reshape(bb, ss, P)
    return (loss, prompt_id_preds, acc, p_counts)


# speedup vs baseline: 1.0758x; 1.0758x over previous
import functools

import jax
import jax.numpy as jnp
from jax import lax
from jax.experimental import pallas as pl
from jax.experimental.pallas import tpu as pltpu
from jax.experimental.pallas import tpu_sc as plsc

B, S, C, P = 64, 32, 8192, 8
N = B * S
R = 256
GRID = N // R
NEG_INF = float("-inf")

L = 16
T_SC = 16
PPT = N * P // T_SC


def _fused_body(x_ref, t_ref, loss_ref, corr_ref, preds_ref):
    i = pl.program_id(0)
    x = x_ref[...]
    t = t_ref[...]
    iota = lax.broadcasted_iota(jnp.int32, (R, C), 1)

    xw = x
    idxs, vals = [], []
    for _ in range(P):
        m = jnp.max(xw, axis=1, keepdims=True)
        cand = jnp.where(xw == m, iota, C)
        idx = jnp.min(cand, axis=1, keepdims=True)
        idxs.append(idx)
        vals.append(m)
        xw = jnp.where(cand == idx, NEG_INF, xw)
    preds_ref[...] = jnp.concatenate(idxs, axis=1)

    rmax = vals[0]
    sumexp = jnp.sum(jnp.exp(xw - rmax), axis=1, keepdims=True)
    for k in range(P):
        sumexp = sumexp + jnp.exp(vals[k] - rmax)
    lse = jnp.log(sumexp) + rmax

    tsum = jnp.zeros((R, 1), jnp.float32)
    for j in range(P):
        tj = t[:, j:j + 1]
        tsum = tsum + jnp.sum(jnp.where(iota == tj, x, 0.0), axis=1,
                              keepdims=True)
    loss_blk = jnp.sum(lse - tsum / jnp.float32(P))

    corr_blk = jnp.float32(0.0)
    for k in range(P):
        mk = jnp.zeros((R, 1), jnp.bool_)
        for j in range(P):
            mk = mk | (idxs[k] == t[:, j:j + 1])
        corr_blk = corr_blk + jnp.sum(jnp.where(mk, 1.0, 0.0))

    @pl.when(i == 0)
    def _():
        loss_ref[...] = jnp.zeros_like(loss_ref)
        corr_ref[...] = jnp.zeros_like(corr_ref)

    loss_ref[...] += loss_blk
    corr_ref[...] += corr_blk


@jax.jit
def _run_tc(x2, t2):
    return pl.pallas_call(
        _fused_body,
        grid=(GRID,),
        in_specs=[
            pl.BlockSpec((R, C), lambda i: (i, 0)),
            pl.BlockSpec((R, P), lambda i: (i, 0)),
        ],
        out_specs=[
            pl.BlockSpec((1, 1), lambda i: (0, 0)),
            pl.BlockSpec((1, 1), lambda i: (0, 0)),
            pl.BlockSpec((R, P), lambda i: (i, 0)),
        ],
        out_shape=[
            jax.ShapeDtypeStruct((1, 1), jnp.float32),
            jax.ShapeDtypeStruct((1, 1), jnp.float32),
            jax.ShapeDtypeStruct((N, P), jnp.int32),
        ],
    )(x2, t2)


@functools.partial(
    pl.kernel,
    mesh=plsc.VectorSubcoreMesh(core_axis_name="c", subcore_axis_name="s",
                                num_cores=1),
    out_type=jax.ShapeDtypeStruct((C,), jnp.float32),
    scratch_types=[
        pltpu.VMEM((PPT,), jnp.int32),
        pltpu.VMEM((8, 128), jnp.int32),
        pltpu.VMEM((128,), jnp.float32),
        pltpu.VMEM((C,), jnp.float32),
        pltpu.VMEM_SHARED((C,), jnp.float32),
        pltpu.SemaphoreType.DMA,
    ],
)
def _sc_hist(preds_hbm, out_hbm, pidx_v, pidx2_v, ones_v, z_v, shared, sem):
    wid = lax.axis_index("s")
    base = wid * PPT
    pltpu.sync_copy(preds_hbm.at[pl.ds(base, PPT)], pidx_v)
    one16 = jnp.ones((L,), jnp.float32)
    zero16 = jnp.zeros((L,), jnp.float32)
    for k in range(8):
        ones_v[pl.ds(k * L, L)] = one16
    for k in range(PPT // L):
        pidx2_v[k // 8, pl.ds((k % 8) * L, L)] = pidx_v[pl.ds(k * L, L)]

    @pl.when(wid == 0)
    def _():
        for k in range(C // L):
            z_v[pl.ds(k * L, L)] = zero16
        pltpu.sync_copy(z_v, shared)

    plsc.subcore_barrier()
    for b in range(8):
        pltpu.sync_copy(ones_v, shared.at[pidx2_v.at[b]], add=True)
    plsc.subcore_barrier()

    @pl.when(wid == 0)
    def _():
        pltpu.sync_copy(shared, out_hbm)


def kernel(output, target):
    bb, ss, cc = output.shape
    x2 = output.reshape(N, C)
    t2 = target.reshape(N, P)
    loss_sum, correct, preds = _run_tc(x2, t2)
    counts = _sc_hist(preds.reshape(N * P))
    loss = loss_sum[0, 0] / jnp.float32(N)
    acc = correct[0, 0] / jnp.float32(N * P) * 100.0
    p_counts = (counts / counts.sum() * 100.0).astype(jnp.int32)
    prompt_id_preds = preds.reshape(bb, ss, P)
    return (loss, prompt_id_preds, acc, p_counts)


# speedup vs baseline: 1.1795x; 1.0964x over previous
import functools

import jax
import jax.numpy as jnp
from jax import lax
from jax.experimental import pallas as pl
from jax.experimental.pallas import tpu as pltpu
from jax.experimental.pallas import tpu_sc as plsc

B, S, C, P = 64, 32, 8192, 8
N = B * S
R = 256
GRID = N // R
NEG_INF = float("-inf")

L = 16
T_SC = 16
PPT = N * P // T_SC


def _fused_body(x_ref, t_ref, loss_ref, corr_ref, preds_ref):
    i = pl.program_id(0)
    x = x_ref[...]
    t = t_ref[...]
    iota = lax.broadcasted_iota(jnp.int32, (R, C), 1)

    xw = x
    idxs, vals = [], []
    for k in range(P):
        m = jnp.max(xw, axis=1, keepdims=True)
        cand = jnp.where(xw == m, iota, C)
        idx = jnp.min(cand, axis=1, keepdims=True)
        idxs.append(idx)
        vals.append(m)
        if k < P - 1:
            xw = jnp.where(cand == idx, NEG_INF, xw)
    preds = jnp.concatenate(idxs, axis=1)
    preds_ref[...] = preds

    rmax = vals[0]
    sumexp = jnp.sum(jnp.exp(xw - rmax), axis=1, keepdims=True)
    for k in range(P - 1):
        sumexp = sumexp + jnp.exp(vals[k] - rmax)
    lse = jnp.log(sumexp) + rmax

    tsum = jnp.zeros((R, 1), jnp.float32)
    for j in range(P):
        tj = t[:, j:j + 1]
        tsum = tsum + jnp.sum(jnp.where(iota == tj, x, 0.0), axis=1,
                              keepdims=True)
    loss_blk = jnp.sum(lse - tsum / jnp.float32(P))

    mk = jnp.zeros((R, P), jnp.bool_)
    for j in range(P):
        mk = mk | (preds == t[:, j:j + 1])
    corr_blk = jnp.sum(jnp.where(mk, 1.0, 0.0))

    @pl.when(i == 0)
    def _():
        loss_ref[...] = jnp.zeros_like(loss_ref)
        corr_ref[...] = jnp.zeros_like(corr_ref)

    loss_ref[...] += loss_blk
    corr_ref[...] += corr_blk


@jax.jit
def _run_tc(x2, t2):
    return pl.pallas_call(
        _fused_body,
        grid=(GRID,),
        in_specs=[
            pl.BlockSpec((R, C), lambda i: (i, 0)),
            pl.BlockSpec((R, P), lambda i: (i, 0)),
        ],
        out_specs=[
            pl.BlockSpec((1, 1), lambda i: (0, 0)),
            pl.BlockSpec((1, 1), lambda i: (0, 0)),
            pl.BlockSpec((R, P), lambda i: (i, 0)),
        ],
        out_shape=[
            jax.ShapeDtypeStruct((1, 1), jnp.float32),
            jax.ShapeDtypeStruct((1, 1), jnp.float32),
            jax.ShapeDtypeStruct((N, P), jnp.int32),
        ],
    )(x2, t2)


@functools.partial(
    pl.kernel,
    mesh=plsc.VectorSubcoreMesh(core_axis_name="c", subcore_axis_name="s",
                                num_cores=1),
    out_type=jax.ShapeDtypeStruct((C,), jnp.float32),
    scratch_types=[
        pltpu.VMEM((PPT,), jnp.int32),
        pltpu.VMEM((8, 128), jnp.int32),
        pltpu.VMEM((128,), jnp.float32),
        pltpu.VMEM((C,), jnp.float32),
        pltpu.VMEM_SHARED((C,), jnp.float32),
        pltpu.SemaphoreType.DMA,
    ],
)
def _sc_hist(preds_hbm, out_hbm, pidx_v, pidx2_v, ones_v, z_v, shared, sem):
    wid = lax.axis_index("s")
    base = wid * PPT
    pltpu.sync_copy(preds_hbm.at[pl.ds(base, PPT)], pidx_v)
    one16 = jnp.ones((L,), jnp.float32)
    zero16 = jnp.zeros((L,), jnp.float32)
    for k in range(8):
        ones_v[pl.ds(k * L, L)] = one16
    for k in range(PPT // L):
        pidx2_v[k // 8, pl.ds((k % 8) * L, L)] = pidx_v[pl.ds(k * L, L)]

    @pl.when(wid == 0)
    def _():
        for k in range(C // L):
            z_v[pl.ds(k * L, L)] = zero16
        pltpu.sync_copy(z_v, shared)

    plsc.subcore_barrier()
    for b in range(8):
        pltpu.sync_copy(ones_v, shared.at[pidx2_v.at[b]], add=True)
    plsc.subcore_barrier()

    @pl.when(wid == 0)
    def _():
        pltpu.sync_copy(shared, out_hbm)


def kernel(output, target):
    bb, ss, cc = output.shape
    x2 = output.reshape(N, C)
    t2 = target.reshape(N, P)
    loss_sum, correct, preds = _run_tc(x2, t2)
    counts = _sc_hist(preds.reshape(N * P))
    loss = loss_sum[0, 0] / jnp.float32(N)
    acc = correct[0, 0] / jnp.float32(N * P) * 100.0
    p_counts = (counts / counts.sum() * 100.0).astype(jnp.int32)
    prompt_id_preds = preds.reshape(bb, ss, P)
    return (loss, prompt_id_preds, acc, p_counts)
